# baseline (speedup 1.0000x reference)
import functools

import jax
import jax.numpy as jnp
from jax import lax
from jax.experimental import pallas as pl
from jax.experimental.pallas import tpu as pltpu
from jax.experimental.pallas import tpu_sc as plsc

L = 7
H = 128
IN = 128
OUT = 112
N = 10000
E = 320000
T = 1.0
MSG_EPS = 1e-7
LN_EPS = 1e-5

NC = 2
NS = 16
NT = NC * NS
G = 64
GPT = 160
QG = 40
EPT = G * GPT
EPAD = NT * EPT
ACC_R = 10112
RPT = ACC_R // NS
NBUF = 4

BN = 1000
NB = N // BN

_HIGH = lax.Precision.HIGHEST


def _enc_body(x_ref, w_ref, b_ref, o_ref):
    o_ref[...] = (
        lax.dot_general(x_ref[...], w_ref[...], (((1,), (0,)), ((), ())),
                        precision=_HIGH, preferred_element_type=jnp.float32)
        + b_ref[...])


def _encode(x, w, b):
    return pl.pallas_call(
        _enc_body,
        grid=(NB,),
        in_specs=[
            pl.BlockSpec((BN, IN), lambda i: (i, 0)),
            pl.BlockSpec((IN, H), lambda i: (0, 0)),
            pl.BlockSpec((1, H), lambda i: (0, 0)),
        ],
        out_specs=pl.BlockSpec((BN, H), lambda i: (i, 0)),
        out_shape=jax.ShapeDtypeStruct((N, H), jnp.float32),
    )(x, w, b)


def _stage_body(first, h_ref, g_ref, b_ref, h2_ref, tp_ref, te_ref,
                msg_ref, mx_ref):
    p = pl.program_id(0)
    i = pl.program_id(1)

    @pl.when(p == 0)
    def _():
        hb = h_ref[...]
        if first:
            h2 = hb
            msg = jnp.maximum(hb, 0.0) + MSG_EPS
        else:
            mu = jnp.mean(hb, axis=-1, keepdims=True)
            var = jnp.mean((hb - mu) ** 2, axis=-1, keepdims=True)
            h2 = jnp.maximum(
                (hb - mu) * lax.rsqrt(var + LN_EPS) * g_ref[...] + b_ref[...],
                0.0)
            msg = h2 + MSG_EPS
        h2_ref[...] = h2
        msg_ref[pl.ds(i * BN, BN), :] = msg
        bmax = jnp.max(msg, axis=0, keepdims=True)
        prev = jnp.where(i == 0, 0.0, mx_ref[...])
        mx_ref[...] = jnp.maximum(prev, bmax)

    @pl.when(p == 1)
    def _():
        msg = msg_ref[pl.ds(i * BN, BN), :]
        m_scaled = jnp.max(mx_ref[...], axis=0, keepdims=True) * T
        eh = jnp.exp(msg * T - m_scaled)
        tp_ref[...] = msg * eh
        te_ref[...] = eh
        if first:
            h2_ref[...] = h_ref[...]
        else:
            h2_ref[...] = msg - MSG_EPS


def _stage(h, g, b, first):
    return pl.pallas_call(
        functools.partial(_stage_body, first),
        grid=(2, NB),
        in_specs=[
            pl.BlockSpec((BN, H), lambda p, i: (i, 0)),
            pl.BlockSpec((1, H), lambda p, i: (0, 0)),
            pl.BlockSpec((1, H), lambda p, i: (0, 0)),
        ],
        out_specs=[
            pl.BlockSpec((BN, H), lambda p, i: (i, 0)),
            pl.BlockSpec((BN, H), lambda p, i: (i, 0)),
            pl.BlockSpec((BN, H), lambda p, i: (i, 0)),
        ],
        out_shape=[
            jax.ShapeDtypeStruct((N, H), jnp.float32),
            jax.ShapeDtypeStruct((N, H), jnp.float32),
            jax.ShapeDtypeStruct((N, H), jnp.float32),
        ],
        scratch_shapes=[
            pltpu.VMEM((N, H), jnp.float32),
            pltpu.VMEM((8, H), jnp.float32),
        ],
    )(h, g, b)


def _post_body(first, acc_ref, h2_ref, h_ref, w_ref, b_ref, o_ref):
    num = acc_ref[0, 0] + acc_ref[0, 1]
    den = acc_ref[1, 0] + acc_ref[1, 1]
    m = jnp.where(den > 0.0, num / den, 0.0)
    z = h2_ref[...] + m
    r = (lax.dot_general(z, w_ref[...], (((1,), (0,)), ((), ())),
                         precision=_HIGH, preferred_element_type=jnp.float32)
         + b_ref[...])
    if not first:
        r = r + h_ref[...]
    o_ref[...] = r


def _post(acc, h2, h, w, b, first):
    return pl.pallas_call(
        functools.partial(_post_body, first),
        grid=(NB,),
        in_specs=[
            pl.BlockSpec((2, 2, BN, H), lambda i: (0, 0, i, 0)),
            pl.BlockSpec((BN, H), lambda i: (i, 0)),
            pl.BlockSpec((BN, H), lambda i: (i, 0)),
            pl.BlockSpec((H, H), lambda i: (0, 0)),
            pl.BlockSpec((1, H), lambda i: (0, 0)),
        ],
        out_specs=pl.BlockSpec((BN, H), lambda i: (i, 0)),
        out_shape=jax.ShapeDtypeStruct((N, H), jnp.float32),
    )(acc, h2, h, w, b)


def _pred_body(h_ref, w_ref, b_ref, o_ref):
    o_ref[...] = (
        lax.dot_general(h_ref[...], w_ref[...], (((1,), (0,)), ((), ())),
                        precision=_HIGH, preferred_element_type=jnp.float32)
        + b_ref[...])


def _predict(h, w, b):
    return pl.pallas_call(
        _pred_body,
        grid=(NB,),
        in_specs=[
            pl.BlockSpec((BN, H), lambda i: (i, 0)),
            pl.BlockSpec((H, 128), lambda i: (0, 0)),
            pl.BlockSpec((1, 128), lambda i: (0, 0)),
        ],
        out_specs=pl.BlockSpec((BN, 128), lambda i: (i, 0)),
        out_shape=jax.ShapeDtypeStruct((N, 128), jnp.float32),
    )(h, w, b)


ZR = 16


def _sc_body(tp_hbm, te_hbm, src_hbm, dst_hbm, out_hbm,
             srci, dsti, r0, r1, r2, r3, zbuf, acc,
             g0, g1, g2, g3, s0, s1, s2, s3, zsem):
    c = lax.axis_index("c")
    s = lax.axis_index("s")
    w = c * NS + s
    base = s * RPT
    rows = [r0, r1, r2, r3]
    gsem = [g0, g1, g2, g3]
    ssem = [s0, s1, s2, s3]

    @pl.loop(0, ZR)
    def _(r):
        @pl.loop(0, H, step=16)
        def _(cc):
            zbuf[r, pl.ds(cc, 16)] = jnp.zeros((16,), jnp.float32)

    for phase in range(2):
        tab = tp_hbm if phase == 0 else te_hbm

        for r in range(0, RPT - ZR + 1, ZR):
            pltpu.async_copy(zbuf, acc.at[pl.ds(base + r, ZR)], zsem)
        pltpu.async_copy(zbuf.at[pl.ds(0, RPT % ZR)],
                         acc.at[pl.ds(base + RPT - RPT % ZR, RPT % ZR)], zsem)
        for r in range(0, RPT - ZR + 1, ZR):
            pltpu.make_async_copy(zbuf, acc.at[pl.ds(base, ZR)], zsem).wait()
        pltpu.make_async_copy(zbuf.at[pl.ds(0, RPT % ZR)],
                              acc.at[pl.ds(base, RPT % ZR)], zsem).wait()
        plsc.subcore_barrier()

        for q in range(GPT // QG):
            qbase = w * GPT + q * QG
            pltpu.sync_copy(src_hbm.at[pl.ds(qbase, QG)], srci)
            pltpu.sync_copy(dst_hbm.at[pl.ds(qbase, QG)], dsti)
            for d in range(3):
                pltpu.async_copy(tab.at[srci.at[d]], rows[d], gsem[d])

            @pl.loop(0, QG, step=NBUF)
            def _(g):
                for b in range(NBUF):
                    i = g + b
                    jb = (b + 3) % NBUF
                    pltpu.make_async_copy(tab.at[srci.at[i]], rows[b],
                                          gsem[b]).wait()
                    pltpu.async_copy(rows[b], acc.at[dsti.at[i]], ssem[b],
                                     add=True)

                    @pl.when(i + 3 < QG)
                    def _():
                        @pl.when(i >= 1)
                        def _():
                            pltpu.make_async_copy(
                                rows[jb], acc.at[dsti.at[i]], ssem[jb]).wait()
                        pltpu.async_copy(tab.at[srci.at[i + 3]], rows[jb],
                                         gsem[jb])

            for b in range(NBUF):
                pltpu.make_async_copy(rows[b], acc.at[dsti.at[0]],
                                      ssem[b]).wait()
        plsc.subcore_barrier()

        pltpu.sync_copy(acc.at[pl.ds(base, RPT)],
                        out_hbm.at[phase, c, pl.ds(base, RPT)])
        plsc.subcore_barrier()


def _sc_seg(tab_p, tab_e, src2, dst2):
    mesh = plsc.VectorSubcoreMesh(core_axis_name="c", subcore_axis_name="s")
    kern = pl.kernel(
        _sc_body,
        mesh=mesh,
        out_type=jax.ShapeDtypeStruct((2, NC, ACC_R, H), jnp.float32),
        scratch_types=[
            pltpu.VMEM((QG, G), jnp.int32),
            pltpu.VMEM((QG, G), jnp.int32),
            pltpu.VMEM((G, H), jnp.float32),
            pltpu.VMEM((G, H), jnp.float32),
            pltpu.VMEM((G, H), jnp.float32),
            pltpu.VMEM((G, H), jnp.float32),
            pltpu.VMEM((ZR, H), jnp.float32),
            pltpu.VMEM_SHARED((ACC_R, H), jnp.float32),
            pltpu.SemaphoreType.DMA,
            pltpu.SemaphoreType.DMA,
            pltpu.SemaphoreType.DMA,
            pltpu.SemaphoreType.DMA,
            pltpu.SemaphoreType.DMA,
            pltpu.SemaphoreType.DMA,
            pltpu.SemaphoreType.DMA,
            pltpu.SemaphoreType.DMA,
            pltpu.SemaphoreType.DMA,
        ],
    )
    return kern(tab_p, tab_e, src2, dst2)


def kernel(g_edge_index, x, W_enc, b_enc, W_mlp, b_mlp, gamma, beta,
           W_pred, b_pred):
    src, dst = lax.sort((g_edge_index[0], g_edge_index[1]), num_keys=1)
    pad = EPAD - E
    src2 = jnp.concatenate([src, jnp.zeros((pad,), jnp.int32)]
                           ).reshape(EPAD // G, G)
    dst2 = jnp.concatenate([dst, jnp.full((pad,), N, jnp.int32)]
                           ).reshape(EPAD // G, G)

    h = _encode(x, W_enc, b_enc.reshape(1, H))
    for l in range(L):
        first = (l == 0)
        if first:
            gm = jnp.ones((1, H), jnp.float32)
            bt = jnp.zeros((1, H), jnp.float32)
        else:
            gm = gamma[l - 1].reshape(1, H)
            bt = beta[l - 1].reshape(1, H)
        h2, tab_p, tab_e = _stage(h, gm, bt, first)
        acc = _sc_seg(tab_p, tab_e, src2, dst2)
        h = _post(acc, h2, h, W_mlp[l], b_mlp[l].reshape(1, H), first)

    wp = jnp.pad(W_pred, ((0, 0), (0, 128 - OUT)))
    bp = jnp.pad(b_pred, (0, 128 - OUT)).reshape(1, 128)
    out = _predict(h, wp, bp)
    return out[:, :OUT]


# speedup vs baseline: 1.1011x; 1.1011x over previous
import functools

import jax
import jax.numpy as jnp
from jax import lax
from jax.experimental import pallas as pl
from jax.experimental.pallas import tpu as pltpu
from jax.experimental.pallas import tpu_sc as plsc

L = 7
H = 128
IN = 128
OUT = 112
N = 10000
E = 320000
T = 1.0
MSG_EPS = 1e-7
LN_EPS = 1e-5

NC = 2
NS = 16
NT = NC * NS
G = 64
GPT = 160
QG = 40
EPT = G * GPT
EPAD = NT * EPT
ACC_R = 10112
RPT = ACC_R // NS
NBUF = 4

BN = 1000
NB = N // BN

_HIGH = lax.Precision.HIGHEST


def _enc_body(x_ref, w_ref, b_ref, o_ref):
    o_ref[...] = (
        lax.dot_general(x_ref[...], w_ref[...], (((1,), (0,)), ((), ())),
                        precision=_HIGH, preferred_element_type=jnp.float32)
        + b_ref[...])


def _encode(x, w, b):
    return pl.pallas_call(
        _enc_body,
        grid=(NB,),
        in_specs=[
            pl.BlockSpec((BN, IN), lambda i: (i, 0)),
            pl.BlockSpec((IN, H), lambda i: (0, 0)),
            pl.BlockSpec((1, H), lambda i: (0, 0)),
        ],
        out_specs=pl.BlockSpec((BN, H), lambda i: (i, 0)),
        out_shape=jax.ShapeDtypeStruct((N, H), jnp.float32),
    )(x, w, b)


def _dot(a, b):
    return lax.dot_general(a, b, (((1,), (0,)), ((), ())),
                           precision=_HIGH, preferred_element_type=jnp.float32)


def _tables_p1(i, msg_ref, mx_ref, tp_ref, te_ref):
    msg = msg_ref[pl.ds(i * BN, BN), :]
    m_scaled = jnp.max(mx_ref[...], axis=0, keepdims=True) * T
    eh = jnp.exp(msg * T - m_scaled)
    tp_ref[...] = msg * eh
    te_ref[...] = eh
    return msg


def _maxacc(i, msg, mx_ref):
    bmax = jnp.max(msg, axis=0, keepdims=True)
    prev = jnp.where(i == 0, 0.0, mx_ref[...])
    mx_ref[...] = jnp.maximum(prev, bmax)


def _encstage_body(x_ref, w_ref, b_ref, hn_ref, h2_ref, tp_ref, te_ref,
                   hbuf, msg_ref, mx_ref):
    p = pl.program_id(0)
    i = pl.program_id(1)

    @pl.when(p == 0)
    def _():
        h = _dot(x_ref[...], w_ref[...]) + b_ref[...]
        hn_ref[...] = h
        h2_ref[...] = h
        hbuf[pl.ds(i * BN, BN), :] = h
        msg = jnp.maximum(h, 0.0) + MSG_EPS
        msg_ref[pl.ds(i * BN, BN), :] = msg
        _maxacc(i, msg, mx_ref)

    @pl.when(p == 1)
    def _():
        _tables_p1(i, msg_ref, mx_ref, tp_ref, te_ref)
        h = hbuf[pl.ds(i * BN, BN), :]
        hn_ref[...] = h
        h2_ref[...] = h


def _encstage(x, w, b):
    return pl.pallas_call(
        _encstage_body,
        grid=(2, NB),
        in_specs=[
            pl.BlockSpec((BN, IN), lambda p, i: (i, 0)),
            pl.BlockSpec((IN, H), lambda p, i: (0, 0)),
            pl.BlockSpec((1, H), lambda p, i: (0, 0)),
        ],
        out_specs=[
            pl.BlockSpec((BN, H), lambda p, i: (i, 0)),
            pl.BlockSpec((BN, H), lambda p, i: (i, 0)),
            pl.BlockSpec((BN, H), lambda p, i: (i, 0)),
            pl.BlockSpec((BN, H), lambda p, i: (i, 0)),
        ],
        out_shape=[
            jax.ShapeDtypeStruct((N, H), jnp.float32),
            jax.ShapeDtypeStruct((N, H), jnp.float32),
            jax.ShapeDtypeStruct((N, H), jnp.float32),
            jax.ShapeDtypeStruct((N, H), jnp.float32),
        ],
        scratch_shapes=[
            pltpu.VMEM((N, H), jnp.float32),
            pltpu.VMEM((N, H), jnp.float32),
            pltpu.VMEM((8, H), jnp.float32),
        ],
    )(x, w, b)


def _poststage_body(first, acc_ref, h2_ref, h_ref, w_ref, b_ref, g_ref,
                    bt_ref, hn_ref, h2n_ref, tp_ref, te_ref,
                    hbuf, msg_ref, mx_ref):
    p = pl.program_id(0)
    i = pl.program_id(1)

    @pl.when(p == 0)
    def _():
        num = acc_ref[0, 0] + acc_ref[0, 1]
        den = acc_ref[1, 0] + acc_ref[1, 1]
        m = jnp.where(den > 0.0, num / den, 0.0)
        z = h2_ref[...] + m
        hn = _dot(z, w_ref[...]) + b_ref[...]
        if not first:
            hn = hn + h_ref[...]
        hn_ref[...] = hn
        hbuf[pl.ds(i * BN, BN), :] = hn
        mu = jnp.mean(hn, axis=-1, keepdims=True)
        var = jnp.mean((hn - mu) ** 2, axis=-1, keepdims=True)
        h2n = jnp.maximum(
            (hn - mu) * lax.rsqrt(var + LN_EPS) * g_ref[...] + bt_ref[...],
            0.0)
        h2n_ref[...] = h2n
        msg = h2n + MSG_EPS
        msg_ref[pl.ds(i * BN, BN), :] = msg
        _maxacc(i, msg, mx_ref)

    @pl.when(p == 1)
    def _():
        msg = _tables_p1(i, msg_ref, mx_ref, tp_ref, te_ref)
        hn_ref[...] = hbuf[pl.ds(i * BN, BN), :]
        h2n_ref[...] = msg - MSG_EPS


def _poststage(acc, h2, h, w, b, g, bt, first):
    return pl.pallas_call(
        functools.partial(_poststage_body, first),
        grid=(2, NB),
        in_specs=[
            pl.BlockSpec((2, 2, BN, H), lambda p, i: (0, 0, i, 0)),
            pl.BlockSpec((BN, H), lambda p, i: (i, 0)),
            pl.BlockSpec((BN, H), lambda p, i: (i, 0)),
            pl.BlockSpec((H, H), lambda p, i: (0, 0)),
            pl.BlockSpec((1, H), lambda p, i: (0, 0)),
            pl.BlockSpec((1, H), lambda p, i: (0, 0)),
            pl.BlockSpec((1, H), lambda p, i: (0, 0)),
        ],
        out_specs=[
            pl.BlockSpec((BN, H), lambda p, i: (i, 0)),
            pl.BlockSpec((BN, H), lambda p, i: (i, 0)),
            pl.BlockSpec((BN, H), lambda p, i: (i, 0)),
            pl.BlockSpec((BN, H), lambda p, i: (i, 0)),
        ],
        out_shape=[
            jax.ShapeDtypeStruct((N, H), jnp.float32),
            jax.ShapeDtypeStruct((N, H), jnp.float32),
            jax.ShapeDtypeStruct((N, H), jnp.float32),
            jax.ShapeDtypeStruct((N, H), jnp.float32),
        ],
        scratch_shapes=[
            pltpu.VMEM((N, H), jnp.float32),
            pltpu.VMEM((N, H), jnp.float32),
            pltpu.VMEM((8, H), jnp.float32),
        ],
    )(acc, h2, h, w, b, g, bt)


def _postpred_body(acc_ref, h2_ref, h_ref, w_ref, b_ref, wp_ref, bp_ref,
                   o_ref):
    num = acc_ref[0, 0] + acc_ref[0, 1]
    den = acc_ref[1, 0] + acc_ref[1, 1]
    m = jnp.where(den > 0.0, num / den, 0.0)
    z = h2_ref[...] + m
    hn = _dot(z, w_ref[...]) + b_ref[...] + h_ref[...]
    o_ref[...] = _dot(hn, wp_ref[...]) + bp_ref[...]


def _postpred(acc, h2, h, w, b, wp, bp):
    return pl.pallas_call(
        _postpred_body,
        grid=(NB,),
        in_specs=[
            pl.BlockSpec((2, 2, BN, H), lambda i: (0, 0, i, 0)),
            pl.BlockSpec((BN, H), lambda i: (i, 0)),
            pl.BlockSpec((BN, H), lambda i: (i, 0)),
            pl.BlockSpec((H, H), lambda i: (0, 0)),
            pl.BlockSpec((1, H), lambda i: (0, 0)),
            pl.BlockSpec((H, 128), lambda i: (0, 0)),
            pl.BlockSpec((1, 128), lambda i: (0, 0)),
        ],
        out_specs=pl.BlockSpec((BN, 128), lambda i: (i, 0)),
        out_shape=jax.ShapeDtypeStruct((N, 128), jnp.float32),
    )(acc, h2, h, w, b, wp, bp)


ZR = 16


def _sc_body(tp_hbm, te_hbm, src_hbm, dst_hbm, out_hbm,
             srci, dsti, r0, r1, r2, r3, zbuf, acc,
             g0, g1, g2, g3, s0, s1, s2, s3, zsem):
    c = lax.axis_index("c")
    s = lax.axis_index("s")
    w = c * NS + s
    base = s * RPT
    rows = [r0, r1, r2, r3]
    gsem = [g0, g1, g2, g3]
    ssem = [s0, s1, s2, s3]

    @pl.loop(0, ZR)
    def _(r):
        @pl.loop(0, H, step=16)
        def _(cc):
            zbuf[r, pl.ds(cc, 16)] = jnp.zeros((16,), jnp.float32)

    for phase in range(2):
        tab = tp_hbm if phase == 0 else te_hbm

        for r in range(0, RPT - ZR + 1, ZR):
            pltpu.async_copy(zbuf, acc.at[pl.ds(base + r, ZR)], zsem)
        pltpu.async_copy(zbuf.at[pl.ds(0, RPT % ZR)],
                         acc.at[pl.ds(base + RPT - RPT % ZR, RPT % ZR)], zsem)
        for r in range(0, RPT - ZR + 1, ZR):
            pltpu.make_async_copy(zbuf, acc.at[pl.ds(base, ZR)], zsem).wait()
        pltpu.make_async_copy(zbuf.at[pl.ds(0, RPT % ZR)],
                              acc.at[pl.ds(base, RPT % ZR)], zsem).wait()
        plsc.subcore_barrier()

        for q in range(GPT // QG):
            qbase = w * GPT + q * QG
            pltpu.sync_copy(src_hbm.at[pl.ds(qbase, QG)], srci)
            pltpu.sync_copy(dst_hbm.at[pl.ds(qbase, QG)], dsti)
            for d in range(3):
                pltpu.async_copy(tab.at[srci.at[d]], rows[d], gsem[d])

            @pl.loop(0, QG, step=NBUF)
            def _(g):
                for b in range(NBUF):
                    i = g + b
                    jb = (b + 3) % NBUF
                    pltpu.make_async_copy(tab.at[srci.at[i]], rows[b],
                                          gsem[b]).wait()
                    pltpu.async_copy(rows[b], acc.at[dsti.at[i]], ssem[b],
                                     add=True)

                    @pl.when(i + 3 < QG)
                    def _():
                        @pl.when(i >= 1)
                        def _():
                            pltpu.make_async_copy(
                                rows[jb], acc.at[dsti.at[i]], ssem[jb]).wait()
                        pltpu.async_copy(tab.at[srci.at[i + 3]], rows[jb],
                                         gsem[jb])

            for b in range(NBUF):
                pltpu.make_async_copy(rows[b], acc.at[dsti.at[0]],
                                      ssem[b]).wait()
        plsc.subcore_barrier()

        pltpu.sync_copy(acc.at[pl.ds(base, RPT)],
                        out_hbm.at[phase, c, pl.ds(base, RPT)])


def _sc_seg(tab_p, tab_e, src2, dst2):
    mesh = plsc.VectorSubcoreMesh(core_axis_name="c", subcore_axis_name="s")
    kern = pl.kernel(
        _sc_body,
        mesh=mesh,
        out_type=jax.ShapeDtypeStruct((2, NC, ACC_R, H), jnp.float32),
        scratch_types=[
            pltpu.VMEM((QG, G), jnp.int32),
            pltpu.VMEM((QG, G), jnp.int32),
            pltpu.VMEM((G, H), jnp.float32),
            pltpu.VMEM((G, H), jnp.float32),
            pltpu.VMEM((G, H), jnp.float32),
            pltpu.VMEM((G, H), jnp.float32),
            pltpu.VMEM((ZR, H), jnp.float32),
            pltpu.VMEM_SHARED((ACC_R, H), jnp.float32),
            pltpu.SemaphoreType.DMA,
            pltpu.SemaphoreType.DMA,
            pltpu.SemaphoreType.DMA,
            pltpu.SemaphoreType.DMA,
            pltpu.SemaphoreType.DMA,
            pltpu.SemaphoreType.DMA,
            pltpu.SemaphoreType.DMA,
            pltpu.SemaphoreType.DMA,
            pltpu.SemaphoreType.DMA,
        ],
    )
    return kern(tab_p, tab_e, src2, dst2)


def kernel(g_edge_index, x, W_enc, b_enc, W_mlp, b_mlp, gamma, beta,
           W_pred, b_pred):
    src = g_edge_index[0]
    dst = g_edge_index[1]
    pad = EPAD - E
    src2 = jnp.concatenate([src, jnp.zeros((pad,), jnp.int32)]
                           ).reshape(EPAD // G, G)
    dst2 = jnp.concatenate([dst, jnp.full((pad,), N, jnp.int32)]
                           ).reshape(EPAD // G, G)

    h, h2, tab_p, tab_e = _encstage(x, W_enc, b_enc.reshape(1, H))
    for l in range(L - 1):
        acc = _sc_seg(tab_p, tab_e, src2, dst2)
        h, h2, tab_p, tab_e = _poststage(
            acc, h2, h, W_mlp[l], b_mlp[l].reshape(1, H),
            gamma[l].reshape(1, H), beta[l].reshape(1, H), l == 0)

    acc = _sc_seg(tab_p, tab_e, src2, dst2)
    wp = jnp.pad(W_pred, ((0, 0), (0, 128 - OUT)))
    bp = jnp.pad(b_pred, (0, 128 - OUT)).reshape(1, 128)
    out = _postpred(acc, h2, h, W_mlp[L - 1], b_mlp[L - 1].reshape(1, H),
                    wp, bp)
    return out[:, :OUT]


# speedup vs baseline: 1.1049x; 1.0034x over previous
import functools

import jax
import jax.numpy as jnp
from jax import lax
from jax.experimental import pallas as pl
from jax.experimental.pallas import tpu as pltpu
from jax.experimental.pallas import tpu_sc as plsc

L = 7
H = 128
IN = 128
OUT = 112
N = 10000
E = 320000
T = 1.0
MSG_EPS = 1e-7
LN_EPS = 1e-5

NC = 2
NS = 16
NT = NC * NS
G = 64
GPT = 160
QG = 40
EPT = G * GPT
EPAD = NT * EPT
ACC_R = 10112
RPT = ACC_R // NS
NBUF = 4

BN = 1000
NB = N // BN

_HIGH = lax.Precision.HIGHEST


def _enc_body(x_ref, w_ref, b_ref, o_ref):
    o_ref[...] = (
        lax.dot_general(x_ref[...], w_ref[...], (((1,), (0,)), ((), ())),
                        precision=_HIGH, preferred_element_type=jnp.float32)
        + b_ref[...])


def _encode(x, w, b):
    return pl.pallas_call(
        _enc_body,
        grid=(NB,),
        in_specs=[
            pl.BlockSpec((BN, IN), lambda i: (i, 0)),
            pl.BlockSpec((IN, H), lambda i: (0, 0)),
            pl.BlockSpec((1, H), lambda i: (0, 0)),
        ],
        out_specs=pl.BlockSpec((BN, H), lambda i: (i, 0)),
        out_shape=jax.ShapeDtypeStruct((N, H), jnp.float32),
    )(x, w, b)


def _dot(a, b):
    return lax.dot_general(a, b, (((1,), (0,)), ((), ())),
                           precision=_HIGH, preferred_element_type=jnp.float32)


def _tables_p1(i, msg_ref, mx_ref, tp_ref, te_ref):
    msg = msg_ref[pl.ds(i * BN, BN), :]
    m_scaled = jnp.max(mx_ref[...], axis=0, keepdims=True) * T
    eh = jnp.exp(msg * T - m_scaled)
    tp_ref[...] = msg * eh
    te_ref[...] = eh
    return msg


def _maxacc(i, msg, mx_ref):
    bmax = jnp.max(msg, axis=0, keepdims=True)
    prev = jnp.where(i == 0, 0.0, mx_ref[...])
    mx_ref[...] = jnp.maximum(prev, bmax)


def _encstage_body(x_ref, w_ref, b_ref, hn_ref, h2_ref, tp_ref, te_ref,
                   hbuf, msg_ref, mx_ref):
    p = pl.program_id(0)
    i = pl.program_id(1)

    @pl.when(p == 0)
    def _():
        h = _dot(x_ref[...], w_ref[...]) + b_ref[...]
        hn_ref[...] = h
        h2_ref[...] = h
        hbuf[pl.ds(i * BN, BN), :] = h
        msg = jnp.maximum(h, 0.0) + MSG_EPS
        msg_ref[pl.ds(i * BN, BN), :] = msg
        _maxacc(i, msg, mx_ref)

    @pl.when(p == 1)
    def _():
        _tables_p1(i, msg_ref, mx_ref, tp_ref, te_ref)
        h = hbuf[pl.ds(i * BN, BN), :]
        hn_ref[...] = h
        h2_ref[...] = h


def _encstage(x, w, b):
    return pl.pallas_call(
        _encstage_body,
        grid=(2, NB),
        in_specs=[
            pl.BlockSpec((BN, IN), lambda p, i: (i, 0)),
            pl.BlockSpec((IN, H), lambda p, i: (0, 0)),
            pl.BlockSpec((1, H), lambda p, i: (0, 0)),
        ],
        out_specs=[
            pl.BlockSpec((BN, H), lambda p, i: (i, 0)),
            pl.BlockSpec((BN, H), lambda p, i: (i, 0)),
            pl.BlockSpec((BN, H), lambda p, i: (i, 0)),
            pl.BlockSpec((BN, H), lambda p, i: (i, 0)),
        ],
        out_shape=[
            jax.ShapeDtypeStruct((N, H), jnp.float32),
            jax.ShapeDtypeStruct((N, H), jnp.float32),
            jax.ShapeDtypeStruct((N, H), jnp.float32),
            jax.ShapeDtypeStruct((N, H), jnp.float32),
        ],
        scratch_shapes=[
            pltpu.VMEM((N, H), jnp.float32),
            pltpu.VMEM((N, H), jnp.float32),
            pltpu.VMEM((8, H), jnp.float32),
        ],
    )(x, w, b)


def _poststage_body(first, acc_ref, h2_ref, h_ref, w_ref, b_ref, g_ref,
                    bt_ref, hn_ref, h2n_ref, tp_ref, te_ref,
                    hbuf, msg_ref, mx_ref):
    p = pl.program_id(0)
    i = pl.program_id(1)

    @pl.when(p == 0)
    def _():
        num = acc_ref[0, 0] + acc_ref[0, 1]
        den = acc_ref[1, 0] + acc_ref[1, 1] - num
        m = jnp.where(den > 0.0, num / den, 0.0)
        z = h2_ref[...] + m
        hn = _dot(z, w_ref[...]) + b_ref[...]
        if not first:
            hn = hn + h_ref[...]
        hn_ref[...] = hn
        hbuf[pl.ds(i * BN, BN), :] = hn
        mu = jnp.mean(hn, axis=-1, keepdims=True)
        var = jnp.mean((hn - mu) ** 2, axis=-1, keepdims=True)
        h2n = jnp.maximum(
            (hn - mu) * lax.rsqrt(var + LN_EPS) * g_ref[...] + bt_ref[...],
            0.0)
        h2n_ref[...] = h2n
        msg = h2n + MSG_EPS
        msg_ref[pl.ds(i * BN, BN), :] = msg
        _maxacc(i, msg, mx_ref)

    @pl.when(p == 1)
    def _():
        msg = _tables_p1(i, msg_ref, mx_ref, tp_ref, te_ref)
        hn_ref[...] = hbuf[pl.ds(i * BN, BN), :]
        h2n_ref[...] = msg - MSG_EPS


def _poststage(acc, h2, h, w, b, g, bt, first):
    return pl.pallas_call(
        functools.partial(_poststage_body, first),
        grid=(2, NB),
        in_specs=[
            pl.BlockSpec((2, 2, BN, H), lambda p, i: (0, 0, i, 0)),
            pl.BlockSpec((BN, H), lambda p, i: (i, 0)),
            pl.BlockSpec((BN, H), lambda p, i: (i, 0)),
            pl.BlockSpec((H, H), lambda p, i: (0, 0)),
            pl.BlockSpec((1, H), lambda p, i: (0, 0)),
            pl.BlockSpec((1, H), lambda p, i: (0, 0)),
            pl.BlockSpec((1, H), lambda p, i: (0, 0)),
        ],
        out_specs=[
            pl.BlockSpec((BN, H), lambda p, i: (i, 0)),
            pl.BlockSpec((BN, H), lambda p, i: (i, 0)),
            pl.BlockSpec((BN, H), lambda p, i: (i, 0)),
            pl.BlockSpec((BN, H), lambda p, i: (i, 0)),
        ],
        out_shape=[
            jax.ShapeDtypeStruct((N, H), jnp.float32),
            jax.ShapeDtypeStruct((N, H), jnp.float32),
            jax.ShapeDtypeStruct((N, H), jnp.float32),
            jax.ShapeDtypeStruct((N, H), jnp.float32),
        ],
        scratch_shapes=[
            pltpu.VMEM((N, H), jnp.float32),
            pltpu.VMEM((N, H), jnp.float32),
            pltpu.VMEM((8, H), jnp.float32),
        ],
    )(acc, h2, h, w, b, g, bt)


def _postpred_body(acc_ref, h2_ref, h_ref, w_ref, b_ref, wp_ref, bp_ref,
                   o_ref):
    num = acc_ref[0, 0] + acc_ref[0, 1]
    den = acc_ref[1, 0] + acc_ref[1, 1] - num
    m = jnp.where(den > 0.0, num / den, 0.0)
    z = h2_ref[...] + m
    hn = _dot(z, w_ref[...]) + b_ref[...] + h_ref[...]
    o_ref[...] = _dot(hn, wp_ref[...]) + bp_ref[...]


def _postpred(acc, h2, h, w, b, wp, bp):
    return pl.pallas_call(
        _postpred_body,
        grid=(NB,),
        in_specs=[
            pl.BlockSpec((2, 2, BN, H), lambda i: (0, 0, i, 0)),
            pl.BlockSpec((BN, H), lambda i: (i, 0)),
            pl.BlockSpec((BN, H), lambda i: (i, 0)),
            pl.BlockSpec((H, H), lambda i: (0, 0)),
            pl.BlockSpec((1, H), lambda i: (0, 0)),
            pl.BlockSpec((H, 128), lambda i: (0, 0)),
            pl.BlockSpec((1, 128), lambda i: (0, 0)),
        ],
        out_specs=pl.BlockSpec((BN, 128), lambda i: (i, 0)),
        out_shape=jax.ShapeDtypeStruct((N, 128), jnp.float32),
    )(acc, h2, h, w, b, wp, bp)


ZR = 16


def _sc_body(tp_hbm, te_hbm, src_hbm, dst_hbm, out_hbm,
             srci, dsti, r0, r1, r2, r3, zbuf, acc,
             g0, g1, g2, g3, s0, s1, s2, s3, zsem):
    c = lax.axis_index("c")
    s = lax.axis_index("s")
    w = c * NS + s
    base = s * RPT
    rows = [r0, r1, r2, r3]
    gsem = [g0, g1, g2, g3]
    ssem = [s0, s1, s2, s3]

    @pl.loop(0, ZR)
    def _(r):
        @pl.loop(0, H, step=16)
        def _(cc):
            zbuf[r, pl.ds(cc, 16)] = jnp.zeros((16,), jnp.float32)

    for phase in range(2):
        tab = tp_hbm if phase == 0 else te_hbm

        if phase == 0:
            for r in range(0, RPT - ZR + 1, ZR):
                pltpu.async_copy(zbuf, acc.at[pl.ds(base + r, ZR)], zsem)
            pltpu.async_copy(zbuf.at[pl.ds(0, RPT % ZR)],
                             acc.at[pl.ds(base + RPT - RPT % ZR, RPT % ZR)],
                             zsem)
            for r in range(0, RPT - ZR + 1, ZR):
                pltpu.make_async_copy(zbuf, acc.at[pl.ds(base, ZR)],
                                      zsem).wait()
            pltpu.make_async_copy(zbuf.at[pl.ds(0, RPT % ZR)],
                                  acc.at[pl.ds(base, RPT % ZR)], zsem).wait()
        plsc.subcore_barrier()

        for q in range(GPT // QG):
            qbase = w * GPT + q * QG
            pltpu.sync_copy(src_hbm.at[pl.ds(qbase, QG)], srci)
            pltpu.sync_copy(dst_hbm.at[pl.ds(qbase, QG)], dsti)
            for d in range(3):
                pltpu.async_copy(tab.at[srci.at[d]], rows[d], gsem[d])

            @pl.loop(0, QG, step=NBUF)
            def _(g):
                for b in range(NBUF):
                    i = g + b
                    jb = (b + 3) % NBUF
                    pltpu.make_async_copy(tab.at[srci.at[i]], rows[b],
                                          gsem[b]).wait()
                    pltpu.async_copy(rows[b], acc.at[dsti.at[i]], ssem[b],
                                     add=True)

                    @pl.when(i + 3 < QG)
                    def _():
                        @pl.when(i >= 1)
                        def _():
                            pltpu.make_async_copy(
                                rows[jb], acc.at[dsti.at[i]], ssem[jb]).wait()
                        pltpu.async_copy(tab.at[srci.at[i + 3]], rows[jb],
                                         gsem[jb])

            for b in range(NBUF):
                pltpu.make_async_copy(rows[b], acc.at[dsti.at[0]],
                                      ssem[b]).wait()
        plsc.subcore_barrier()

        pltpu.sync_copy(acc.at[pl.ds(base, RPT)],
                        out_hbm.at[phase, c, pl.ds(base, RPT)])


def _sc_seg(tab_p, tab_e, src2, dst2):
    mesh = plsc.VectorSubcoreMesh(core_axis_name="c", subcore_axis_name="s")
    kern = pl.kernel(
        _sc_body,
        mesh=mesh,
        out_type=jax.ShapeDtypeStruct((2, NC, ACC_R, H), jnp.float32),
        scratch_types=[
            pltpu.VMEM((QG, G), jnp.int32),
            pltpu.VMEM((QG, G), jnp.int32),
            pltpu.VMEM((G, H), jnp.float32),
            pltpu.VMEM((G, H), jnp.float32),
            pltpu.VMEM((G, H), jnp.float32),
            pltpu.VMEM((G, H), jnp.float32),
            pltpu.VMEM((ZR, H), jnp.float32),
            pltpu.VMEM_SHARED((ACC_R, H), jnp.float32),
            pltpu.SemaphoreType.DMA,
            pltpu.SemaphoreType.DMA,
            pltpu.SemaphoreType.DMA,
            pltpu.SemaphoreType.DMA,
            pltpu.SemaphoreType.DMA,
            pltpu.SemaphoreType.DMA,
            pltpu.SemaphoreType.DMA,
            pltpu.SemaphoreType.DMA,
            pltpu.SemaphoreType.DMA,
        ],
    )
    return kern(tab_p, tab_e, src2, dst2)


def kernel(g_edge_index, x, W_enc, b_enc, W_mlp, b_mlp, gamma, beta,
           W_pred, b_pred):
    src = g_edge_index[0]
    dst = g_edge_index[1]
    pad = EPAD - E
    src2 = jnp.concatenate([src, jnp.zeros((pad,), jnp.int32)]
                           ).reshape(EPAD // G, G)
    dst2 = jnp.concatenate([dst, jnp.full((pad,), N, jnp.int32)]
                           ).reshape(EPAD // G, G)

    h, h2, tab_p, tab_e = _encstage(x, W_enc, b_enc.reshape(1, H))
    for l in range(L - 1):
        acc = _sc_seg(tab_p, tab_e, src2, dst2)
        h, h2, tab_p, tab_e = _poststage(
            acc, h2, h, W_mlp[l], b_mlp[l].reshape(1, H),
            gamma[l].reshape(1, H), beta[l].reshape(1, H), l == 0)

    acc = _sc_seg(tab_p, tab_e, src2, dst2)
    wp = jnp.pad(W_pred, ((0, 0), (0, 128 - OUT)))
    bp = jnp.pad(b_pred, (0, 128 - OUT)).reshape(1, 128)
    out = _postpred(acc, h2, h, W_mlp[L - 1], b_mlp[L - 1].reshape(1, H),
                    wp, bp)
    return out[:, :OUT]
